# Initial kernel scaffold; baseline (speedup 1.0000x reference)
import jax
import jax.numpy as jnp
from jax.experimental import pallas as pl


def kernel(x, f, b, W_feat, b_feat, W_tf, W_wf, ln_g, ln_b, W_post, b_post, max_neighbors):
    raise NotImplementedError("write your pallas kernel here")


# speedup vs baseline: 1.5081x; 1.5081x over previous
import jax
import jax.numpy as jnp
from jax import lax
from jax.experimental import pallas as pl
from jax.experimental.pallas import tpu as pltpu
from jax.experimental.pallas import tpu_sc as plsc

N = 10000
D = 128
Q = 1000
K = 16
QP = 1024
NP = 10112
BQ = 8
NC, NS, L = 2, 16, 16
NW = NC * NS
QW = QP // NW
EW = QW * K
IR = EW // 128


def _feat_body(f_ref, w_ref, b_ref, o_ref):
    z = jnp.dot(f_ref[...], w_ref[...], preferred_element_type=jnp.float32)
    z = z + b_ref[...]
    o_ref[...] = z * jax.nn.sigmoid(z)


def _knn_body(qx_ref, xt_ref, nbr_ref, w_ref):
    qx = qx_ref[...]
    d2 = jnp.zeros((BQ, NP), jnp.float32)
    for c in range(3):
        diff = qx[:, c:c + 1] - xt_ref[c:c + 1, :]
        d2 = d2 + diff * diff
    iota = lax.broadcasted_iota(jnp.int32, (BQ, NP), 1)
    big = jnp.float32(3.0e38)
    for j in range(K):
        m = jnp.min(d2, axis=1, keepdims=True)
        idx = jnp.min(jnp.where(d2 == m, iota, NP), axis=1, keepdims=True)
        nbr_ref[:, j:j + 1] = idx
        w_ref[:, j, :] = jnp.broadcast_to(jnp.exp(-m), (BQ, L))
        d2 = jnp.where(iota == idx, big, d2)


def _sc_body(f_hbm, i_hbm, w_hbm, agg_hbm, idx_v, w_v, rows_v, acc_v, sem):
    cid = lax.axis_index("c")
    sid = lax.axis_index("s")
    wid = sid * NC + cid
    qbase = wid * QW
    pltpu.sync_copy(i_hbm.at[pl.ds(wid * IR, IR)], idx_v)
    pltpu.sync_copy(w_hbm.at[pl.ds(wid * EW, EW)], w_v)

    RH = EW // 2
    QH = QW // 2
    for h in range(2):
        cps = [pltpu.async_copy(f_hbm.at[idx_v.at[h * 2 + j]],
                                rows_v.at[pl.ds(j * 128, 128)], sem)
               for j in range(IR // 2)]
        for cp in cps:
            cp.wait()

        def qstep(q, carry):
            accs = [None] * (D // L)
            for k in range(K):
                row = q * K + k
                wk = w_v[row]
                lrow = row - h * RH
                for ci in range(D // L):
                    t = wk * rows_v[lrow, pl.ds(ci * L, L)]
                    accs[ci] = t if k == 0 else accs[ci] + t
            for ci in range(D // L):
                acc_v[q, pl.ds(ci * L, L)] = accs[ci]
            return carry

        lax.fori_loop(h * QH, (h + 1) * QH, qstep, jnp.int32(0))
    pltpu.sync_copy(acc_v, agg_hbm.at[pl.ds(qbase, QW)])


def _head_body(a_ref, wtf_ref, wwf_ref, g_ref, bb_ref, wp_ref, bp_ref,
               tf_ref, w_ref):
    a = a_ref[...]
    tf_ref[...] = jnp.dot(a, wtf_ref[...], preferred_element_type=jnp.float32)
    wf = jnp.dot(a, wwf_ref[...], preferred_element_type=jnp.float32)
    mu = jnp.mean(wf, axis=1, keepdims=True)
    xc = wf - mu
    var = jnp.mean(xc * xc, axis=1, keepdims=True)
    h = xc * lax.rsqrt(var + 1e-5) * g_ref[...] + bb_ref[...]
    h = h * jax.nn.sigmoid(h)
    z = jnp.sum(h * wp_ref[...], axis=1, keepdims=True) + bp_ref[...]
    w_ref[...] = jax.nn.sigmoid(z)


def kernel(x, f, b, W_feat, b_feat, W_tf, W_wf, ln_g, ln_b, W_post, b_post,
           max_neighbors):
    qx = x[::N // Q]
    qx_pad = jnp.concatenate([qx, jnp.zeros((QP - Q, 3), jnp.float32)], axis=0)
    xt = jnp.concatenate(
        [x.T, jnp.full((3, NP - N), 1e30, jnp.float32)], axis=1)

    feats = pl.pallas_call(
        _feat_body,
        grid=(10,),
        in_specs=[pl.BlockSpec((N // 10, D), lambda i: (i, 0)),
                  pl.BlockSpec((D, D), lambda i: (0, 0)),
                  pl.BlockSpec((1, D), lambda i: (0, 0))],
        out_specs=pl.BlockSpec((N // 10, D), lambda i: (i, 0)),
        out_shape=jax.ShapeDtypeStruct((N, D), jnp.float32),
    )(f, W_feat, b_feat.reshape(1, D))

    nbr, wed = pl.pallas_call(
        _knn_body,
        grid=(QP // BQ,),
        in_specs=[pl.BlockSpec((BQ, 3), lambda i: (i, 0)),
                  pl.BlockSpec((3, NP), lambda i: (0, 0))],
        out_specs=[pl.BlockSpec((BQ, K), lambda i: (i, 0)),
                   pl.BlockSpec((BQ, K, L), lambda i: (i, 0, 0))],
        out_shape=[jax.ShapeDtypeStruct((QP, K), jnp.int32),
                   jax.ShapeDtypeStruct((QP, K, L), jnp.float32)],
    )(qx_pad, xt)

    nbr2d = nbr.reshape(QP * K // 128, 128)

    agg = pl.kernel(
        _sc_body,
        out_type=jax.ShapeDtypeStruct((QP, D), jnp.float32),
        mesh=plsc.VectorSubcoreMesh(core_axis_name="c", subcore_axis_name="s",
                                    num_cores=NC, num_subcores=NS),
        scratch_types=[
            pltpu.VMEM((IR, 128), jnp.int32),
            pltpu.VMEM((EW, L), jnp.float32),
            pltpu.VMEM((EW // 2, D), jnp.float32),
            pltpu.VMEM((QW, D), jnp.float32),
            pltpu.SemaphoreType.DMA,
        ],
    )(feats, nbr2d, wed.reshape(QP * K, L))

    tf_out, wcol = pl.pallas_call(
        _head_body,
        out_shape=[jax.ShapeDtypeStruct((QP, D), jnp.float32),
                   jax.ShapeDtypeStruct((QP, 1), jnp.float32)],
    )(agg, W_tf, W_wf, ln_g.reshape(1, D), ln_b.reshape(1, D),
      W_post.reshape(1, D), b_post.reshape(1, 1))

    return qx, tf_out[:Q], wcol[:Q, 0]


# speedup vs baseline: 7.2465x; 4.8052x over previous
import jax
import jax.numpy as jnp
from jax import lax
from jax.experimental import pallas as pl
from jax.experimental.pallas import tpu as pltpu
from jax.experimental.pallas import tpu_sc as plsc

N = 10000
D = 128
Q = 1000
K = 16
QP = 1024
NP = 10112
BQ = 8
NC, NS, L = 2, 16, 16
NW = NC * NS
QW = QP // NW
EW = QW * K
IR = EW // 128


def _feat_body(f_ref, w_ref, b_ref, o_ref):
    z = jnp.dot(f_ref[...], w_ref[...], preferred_element_type=jnp.float32)
    z = z + b_ref[...]
    o_ref[...] = z * jax.nn.sigmoid(z)


T = 5
G = 4
NCH = NP // 128
T2 = 6
CAND = T2 * 128
BQ2 = 128


def _knncand_body(qx_ref, xt_ref, cd_ref, ci_ref):
    qx = qx_ref[...]
    big = jnp.float32(3.0e38)
    bigi = jnp.int32(1 << 20)
    M = [[jnp.full((BQ, 128), big, jnp.float32) for _ in range(T)]
         for _ in range(G)]
    A = [[jnp.full((BQ, 128), bigi, jnp.int32) for _ in range(T)]
         for _ in range(G)]
    for c in range(NCH):
        g = c % G
        sl = slice(c * 128, (c + 1) * 128)
        d = jnp.zeros((BQ, 128), jnp.float32)
        for k in range(3):
            diff = qx[:, k:k + 1] - xt_ref[k:k + 1, sl]
            d = d + diff * diff
        e, ea = d, jnp.full((BQ, 128), c, jnp.int32)
        Mg, Ag = M[g], A[g]
        for t in range(T):
            swap = e < Mg[t]
            Mg[t], e = jnp.where(swap, e, Mg[t]), jnp.where(swap, Mg[t], e)
            Ag[t], ea = jnp.where(swap, ea, Ag[t]), jnp.where(swap, Ag[t], ea)
    M2 = [jnp.full((BQ, 128), big, jnp.float32) for _ in range(T2)]
    A2 = [jnp.full((BQ, 128), bigi, jnp.int32) for _ in range(T2)]
    for g in range(G):
        for t in range(T):
            e, ea = M[g][t], A[g][t]
            for t2 in range(T2):
                swap = e < M2[t2]
                M2[t2], e = (jnp.where(swap, e, M2[t2]),
                             jnp.where(swap, M2[t2], e))
                A2[t2], ea = (jnp.where(swap, ea, A2[t2]),
                              jnp.where(swap, A2[t2], ea))
    cd_ref[...] = jnp.concatenate(M2, axis=1)
    ci_ref[...] = jnp.concatenate(A2, axis=1)


def _knnsel_body(cd_ref, ci_ref, nbr_ref, w_ref):
    Ms = cd_ref[...]
    slotmod = jnp.bitwise_and(
        lax.broadcasted_iota(jnp.int32, (BQ2, CAND), 1), 127)
    As = ci_ref[...] * 128 + slotmod
    big = jnp.float32(3.0e38)
    bigi = jnp.int32(1 << 30)
    for j in range(K):
        m = jnp.min(Ms, axis=1, keepdims=True)
        idx = jnp.min(jnp.where(Ms == m, As, bigi), axis=1,
                      keepdims=True)
        nbr_ref[:, j:j + 1] = idx
        w_ref[:, j, :] = jnp.broadcast_to(jnp.exp(-m), (BQ2, L))
        Ms = jnp.where(As == idx, big, Ms)


def _sc_body(f_hbm, i_hbm, w_hbm, agg_hbm, idx_v, w_v, rows_v, acc_v,
             sem0, sem1):
    cid = lax.axis_index("c")
    sid = lax.axis_index("s")
    wid = sid * NC + cid
    qbase = wid * QW
    pltpu.sync_copy(i_hbm.at[pl.ds(wid * IR, IR)], idx_v)
    pltpu.sync_copy(w_hbm.at[pl.ds(wid * EW, EW)], w_v)

    RH = 128
    QH = RH // K
    sems = [sem0, sem1]
    cps = [None, None]
    cps[0] = pltpu.async_copy(f_hbm.at[idx_v.at[0]], rows_v.at[0], sem0)
    for h in range(IR):
        b = h % 2
        if h + 1 < IR:
            cps[1 - b] = pltpu.async_copy(f_hbm.at[idx_v.at[h + 1]],
                                          rows_v.at[1 - b], sems[1 - b])
        cps[b].wait()

        def qstep(q, carry):
            accs = [None] * (D // L)
            for k in range(K):
                row = q * K + k
                wk = w_v[row]
                lrow = row - h * RH
                for ci in range(D // L):
                    t = wk * rows_v[b, lrow, pl.ds(ci * L, L)]
                    accs[ci] = t if k == 0 else accs[ci] + t
            for ci in range(D // L):
                acc_v[q, pl.ds(ci * L, L)] = accs[ci]
            return carry

        lax.fori_loop(h * QH, (h + 1) * QH, qstep, jnp.int32(0))
    pltpu.sync_copy(acc_v, agg_hbm.at[pl.ds(qbase, QW)])


def _head_body(a_ref, wtf_ref, wwf_ref, g_ref, bb_ref, wp_ref, bp_ref,
               tf_ref, w_ref):
    a = a_ref[...]
    tf_ref[...] = jnp.dot(a, wtf_ref[...], preferred_element_type=jnp.float32)
    wf = jnp.dot(a, wwf_ref[...], preferred_element_type=jnp.float32)
    mu = jnp.mean(wf, axis=1, keepdims=True)
    xc = wf - mu
    var = jnp.mean(xc * xc, axis=1, keepdims=True)
    h = xc * lax.rsqrt(var + 1e-5) * g_ref[...] + bb_ref[...]
    h = h * jax.nn.sigmoid(h)
    z = jnp.sum(h * wp_ref[...], axis=1, keepdims=True) + bp_ref[...]
    w_ref[...] = jax.nn.sigmoid(z)


def kernel(x, f, b, W_feat, b_feat, W_tf, W_wf, ln_g, ln_b, W_post, b_post,
           max_neighbors):
    qx = x[::N // Q]
    qx_pad = jnp.concatenate([qx, jnp.zeros((QP - Q, 3), jnp.float32)], axis=0)
    xt = jnp.concatenate(
        [x.T, jnp.full((3, NP - N), 1e30, jnp.float32)], axis=1)

    feats = pl.pallas_call(
        _feat_body,
        grid=(10,),
        in_specs=[pl.BlockSpec((N // 10, D), lambda i: (i, 0)),
                  pl.BlockSpec((D, D), lambda i: (0, 0)),
                  pl.BlockSpec((1, D), lambda i: (0, 0))],
        out_specs=pl.BlockSpec((N // 10, D), lambda i: (i, 0)),
        out_shape=jax.ShapeDtypeStruct((N, D), jnp.float32),
    )(f, W_feat, b_feat.reshape(1, D))

    cd, ci = pl.pallas_call(
        _knncand_body,
        grid=(QP // BQ,),
        in_specs=[pl.BlockSpec((BQ, 3), lambda i: (i, 0)),
                  pl.BlockSpec((3, NP), lambda i: (0, 0))],
        out_specs=[pl.BlockSpec((BQ, CAND), lambda i: (i, 0)),
                   pl.BlockSpec((BQ, CAND), lambda i: (i, 0))],
        out_shape=[jax.ShapeDtypeStruct((QP, CAND), jnp.float32),
                   jax.ShapeDtypeStruct((QP, CAND), jnp.int32)],
    )(qx_pad, xt)

    nbr, wed = pl.pallas_call(
        _knnsel_body,
        grid=(QP // BQ2,),
        in_specs=[pl.BlockSpec((BQ2, CAND), lambda i: (i, 0)),
                  pl.BlockSpec((BQ2, CAND), lambda i: (i, 0))],
        out_specs=[pl.BlockSpec((BQ2, K), lambda i: (i, 0)),
                   pl.BlockSpec((BQ2, K, L), lambda i: (i, 0, 0))],
        out_shape=[jax.ShapeDtypeStruct((QP, K), jnp.int32),
                   jax.ShapeDtypeStruct((QP, K, L), jnp.float32)],
    )(cd, ci)

    nbr2d = nbr.reshape(QP * K // 128, 128)

    agg = pl.kernel(
        _sc_body,
        out_type=jax.ShapeDtypeStruct((QP, D), jnp.float32),
        mesh=plsc.VectorSubcoreMesh(core_axis_name="c", subcore_axis_name="s",
                                    num_cores=NC, num_subcores=NS),
        scratch_types=[
            pltpu.VMEM((IR, 128), jnp.int32),
            pltpu.VMEM((EW, L), jnp.float32),
            pltpu.VMEM((2, 128, D), jnp.float32),
            pltpu.VMEM((QW, D), jnp.float32),
            pltpu.SemaphoreType.DMA,
            pltpu.SemaphoreType.DMA,
        ],
    )(feats, nbr2d, wed.reshape(QP * K, L))

    tf_out, wcol = pl.pallas_call(
        _head_body,
        out_shape=[jax.ShapeDtypeStruct((QP, D), jnp.float32),
                   jax.ShapeDtypeStruct((QP, 1), jnp.float32)],
    )(agg, W_tf, W_wf, ln_g.reshape(1, D), ln_b.reshape(1, D),
      W_post.reshape(1, D), b_post.reshape(1, 1))

    return qx, tf_out[:Q], wcol[:Q, 0]
